# baseline (speedup 1.0000x reference)
import functools

import jax
import jax.numpy as jnp
from jax import lax
from jax.experimental import pallas as pl
from jax.experimental.pallas import tpu as pltpu
from jax.experimental.pallas import tpu_sc as plsc

F = 8
CH = 128
K = 9
NTILES = 32


def _edge_pass_kernel(n_pad, e_pad):
  ew = e_pad // NTILES
  nch = ew // CH
  nblk = nch // K
  rows_per_tile = n_pad // 16
  max_row = e_pad // CH - K
  mesh = plsc.VectorSubcoreMesh(core_axis_name="c", subcore_axis_name="s")

  @functools.partial(
      pl.kernel,
      out_type=jax.ShapeDtypeStruct((2, n_pad, F), jnp.float32),
      mesh=mesh,
      scratch_types=[
          pltpu.VMEM((K, 2, CH), jnp.int32),
          pltpu.VMEM((K, 2, CH), jnp.int32),
          pltpu.VMEM((K, CH, F), jnp.float32),
          pltpu.VMEM_SHARED((n_pad, F), jnp.float32),
          pltpu.SemaphoreType.DMA((K,)),
          pltpu.SemaphoreType.DMA,
          pltpu.SemaphoreType.DMA,
          pltpu.SemaphoreType.DMA,
      ],
      compiler_params=pltpu.CompilerParams(use_tc_tiling_on_sc=False),
  )
  def edge_pass(t_hbm, idx_hbm, zeros_hbm, out_hbm,
                bufA, bufB, rows, acc, gsem, ssem, isemA, isemB):
    c = lax.axis_index("c")
    s = lax.axis_index("s")
    wid = s * 2 + c
    r0 = s * rows_per_tile
    pltpu.sync_copy(zeros_hbm.at[pl.ds(r0, rows_per_tile)],
                    acc.at[pl.ds(r0, rows_per_tile)])
    plsc.subcore_barrier()

    base_row = wid * nch
    pltpu.sync_copy(idx_hbm.at[pl.ds(base_row, K)], bufA)

    def process(buf):
      gathers = [
          pltpu.async_copy(t_hbm.at[buf.at[j, 0]], rows.at[j], gsem.at[j])
          for j in range(K)
      ]
      scatters = []
      for j in range(K):
        gathers[j].wait()
        scatters.append(
            pltpu.async_copy(rows.at[j], acc.at[buf.at[j, 1]],
                             ssem, add=True))
      for d in scatters:
        d.wait()

    def body(bp, carry):
      rowB = base_row + (2 * bp + 1) * K
      rowC = jnp.minimum(base_row + (2 * bp + 2) * K, max_row)
      iB = pltpu.async_copy(idx_hbm.at[pl.ds(rowB, K)], bufB, isemB)
      process(bufA)
      iB.wait()
      iC = pltpu.async_copy(idx_hbm.at[pl.ds(rowC, K)], bufA, isemA)
      process(bufB)
      iC.wait()
      return carry

    lax.fori_loop(0, nblk // 2, body, 0)
    plsc.subcore_barrier()
    pltpu.sync_copy(acc.at[pl.ds(r0, rows_per_tile)],
                    out_hbm.at[c].at[pl.ds(r0, rows_per_tile)])

  return edge_pass


def _deg_pass_kernel(n_pad, e_pad):
  ew = e_pad // NTILES
  nch = ew // CH
  nblk = nch // K
  slice_per_tile = n_pad // 16
  mesh = plsc.VectorSubcoreMesh(core_axis_name="c", subcore_axis_name="s")

  @functools.partial(
      pl.kernel,
      out_type=jax.ShapeDtypeStruct((2, n_pad), jnp.float32),
      mesh=mesh,
      scratch_types=[
          pltpu.VMEM((K, 2, CH), jnp.int32),
          pltpu.VMEM((CH,), jnp.float32),
          pltpu.VMEM_SHARED((n_pad,), jnp.float32),
          pltpu.SemaphoreType.DMA,
      ],
      compiler_params=pltpu.CompilerParams(use_tc_tiling_on_sc=False),
  )
  def deg_pass(idx_hbm, zeros_hbm, out_hbm, blk, ones, acc, ssem):
    c = lax.axis_index("c")
    s = lax.axis_index("s")
    wid = s * 2 + c
    r0 = s * slice_per_tile
    for j in range(CH // 16):
      ones[pl.ds(j * 16, 16)] = jnp.ones((16,), jnp.float32)
    pltpu.sync_copy(zeros_hbm.at[pl.ds(r0, slice_per_tile)],
                    acc.at[pl.ds(r0, slice_per_tile)])
    plsc.subcore_barrier()

    base_row = wid * nch

    def body(bi, carry):
      row0 = base_row + bi * K
      pltpu.sync_copy(idx_hbm.at[pl.ds(row0, K)], blk)
      scatters = [
          pltpu.async_copy(ones, acc.at[blk.at[j, 1]], ssem, add=True)
          for j in range(K)
      ]
      for d in scatters:
        d.wait()
      return carry

    lax.fori_loop(0, nblk, body, 0)
    plsc.subcore_barrier()
    pltpu.sync_copy(acc.at[pl.ds(r0, slice_per_tile)],
                    out_hbm.at[c].at[pl.ds(r0, slice_per_tile)])

  return deg_pass


def _kdeg_body(p_ref, o_ref):
  deg = p_ref[0] + p_ref[1] + 1.0
  o_ref[...] = lax.rsqrt(jnp.maximum(deg, 1e-12))


def _k1_body(x_ref, w_ref, d_ref, o_ref):
  t = jnp.dot(x_ref[...], w_ref[...], preferred_element_type=jnp.float32)
  o_ref[...] = t * d_ref[...]


def _klayer_body(p_ref, tp_ref, d_ref, w_ref, b_ref, o_ref):
  d = d_ref[...]
  agg = d * (p_ref[0] + p_ref[1] + tp_ref[...])
  h = jnp.maximum(agg + b_ref[...], 0.0)
  o_ref[...] = jnp.dot(h, w_ref[...], preferred_element_type=jnp.float32) * d


def _klast_body(p_ref, tp_ref, d_ref, b_ref, o_ref):
  d = d_ref[...]
  o_ref[...] = d * (p_ref[0] + p_ref[1] + tp_ref[...]) + b_ref[...]


def _padW(w, rin, rout):
  return jnp.pad(w, ((0, rin - w.shape[0]), (0, rout - w.shape[1])))


def _padb(b, r):
  return jnp.pad(b, (0, r - b.shape[0]))


def kernel(x, edge_index, W1, b1, W2, b2, W3, b3, W4, b4, W5, b5,
           W6, b6, W7, b7, W8, b8, W9, b9, W10, b10):
  n = x.shape[0]
  e = edge_index.shape[1]
  grain = NTILES * CH * K * 2
  e_pad = ((e + grain - 1) // grain) * grain
  n_pad = ((n + 2047) // 2048) * 2048
  nr = n_pad * F // 128

  src = edge_index[0]
  dst = edge_index[1]
  pad = e_pad - e
  if pad:
    src = jnp.concatenate([src, jnp.zeros((pad,), jnp.int32)])
    dst = jnp.concatenate([dst, jnp.full((pad,), n, jnp.int32)])
  idxg = jnp.stack([src.reshape(e_pad // CH, CH),
                    dst.reshape(e_pad // CH, CH)], axis=1)
  zeros8 = jnp.zeros((n_pad, F), jnp.float32)
  zeros1 = jnp.zeros((n_pad,), jnp.float32)

  deg_pass = _deg_pass_kernel(n_pad, e_pad)
  edge_pass = _edge_pass_kernel(n_pad, e_pad)

  degp = deg_pass(idxg, zeros1)
  dinv = pl.pallas_call(
      _kdeg_body,
      out_shape=jax.ShapeDtypeStruct((n_pad // 128, 128), jnp.float32),
  )(degp.reshape(2, n_pad // 128, 128))
  dexp = jnp.repeat(dinv.reshape(-1), F)
  d128 = dexp.reshape(nr, 128)

  xp = jnp.pad(x, ((0, n_pad - n), (0, 16 - x.shape[1])))
  Wbd1 = jnp.kron(jnp.eye(8, dtype=jnp.float32), _padW(W1, 16, F))
  tp = pl.pallas_call(
      _k1_body,
      out_shape=jax.ShapeDtypeStruct((n_pad // 8, 8 * F), jnp.float32),
  )(xp.reshape(n_pad // 8, 128), Wbd1, dexp.reshape(n_pad // 8, 8 * F))
  tp = tp.reshape(n_pad, F)

  Ws = [W2, W3, W4, W5, W6, W7, W8, W9, W10]
  bs = [b1, b2, b3, b4, b5, b6, b7, b8, b9]
  eye16 = jnp.eye(16, dtype=jnp.float32)
  for i in range(9):
    p = edge_pass(tp, idxg, zeros8)
    Wbd = jnp.kron(eye16, _padW(Ws[i], F, F))
    bt = jnp.tile(_padb(bs[i], F), 16).reshape(1, 128)
    tp = pl.pallas_call(
        _klayer_body,
        out_shape=jax.ShapeDtypeStruct((nr, 128), jnp.float32),
    )(p.reshape(2, nr, 128), tp.reshape(nr, 128), d128, Wbd, bt)
    tp = tp.reshape(n_pad, F)

  p = edge_pass(tp, idxg, zeros8)
  bt10 = jnp.tile(_padb(b10, F), 16).reshape(1, 128)
  out = pl.pallas_call(
      _klast_body,
      out_shape=jax.ShapeDtypeStruct((nr, 128), jnp.float32),
  )(p.reshape(2, nr, 128), tp.reshape(nr, 128), d128, bt10)
  return out.reshape(n_pad, F)[:n, :1]


# speedup vs baseline: 1.6972x; 1.6972x over previous
import functools

import jax
import jax.numpy as jnp
from jax import lax
from jax.experimental import pallas as pl
from jax.experimental.pallas import tpu as pltpu
from jax.experimental.pallas import tpu_sc as plsc

F = 8
CH = 128
K = 17
NTILES = 32


def _edge_pass_kernel(n_pad, e_pad):
  ew = e_pad // NTILES
  nch = ew // CH
  nblk = nch // K
  rows_per_tile = n_pad // 16
  mesh = plsc.VectorSubcoreMesh(core_axis_name="c", subcore_axis_name="s")

  @functools.partial(
      pl.kernel,
      out_type=jax.ShapeDtypeStruct((2, n_pad, F), jnp.float32),
      mesh=mesh,
      scratch_types=[
          pltpu.VMEM((K, 2, CH), jnp.int32),
          pltpu.VMEM((K, CH, F), jnp.float32),
          pltpu.VMEM_SHARED((n_pad, F), jnp.float32),
          pltpu.SemaphoreType.DMA((K,)),
          pltpu.SemaphoreType.DMA,
      ],
      compiler_params=pltpu.CompilerParams(use_tc_tiling_on_sc=False),
  )
  def edge_pass(t_hbm, idx_hbm, zeros_hbm, out_hbm,
                buf, rows, acc, gsem, ssem):
    c = lax.axis_index("c")
    s = lax.axis_index("s")
    wid = s * 2 + c
    r0 = s * rows_per_tile
    pltpu.sync_copy(zeros_hbm.at[pl.ds(r0, rows_per_tile)],
                    acc.at[pl.ds(r0, rows_per_tile)])
    plsc.subcore_barrier()

    base_row = wid * nch

    def body(bi, carry):
      row0 = base_row + bi * K
      pltpu.sync_copy(idx_hbm.at[pl.ds(row0, K)], buf)
      gathers = [
          pltpu.async_copy(t_hbm.at[buf.at[j, 0]], rows.at[j], gsem.at[j])
          for j in range(K)
      ]
      scatters = []
      for j in range(K):
        gathers[j].wait()
        scatters.append(
            pltpu.async_copy(rows.at[j], acc.at[buf.at[j, 1]],
                             ssem, add=True))
      for d in scatters:
        d.wait()
      return carry

    lax.fori_loop(0, nblk, body, 0)
    plsc.subcore_barrier()
    pltpu.sync_copy(acc.at[pl.ds(r0, rows_per_tile)],
                    out_hbm.at[c].at[pl.ds(r0, rows_per_tile)])

  return edge_pass


def _deg_pass_kernel(n_pad, e_pad):
  ew = e_pad // NTILES
  nch = ew // CH
  nblk = nch // K
  slice_per_tile = n_pad // 16
  mesh = plsc.VectorSubcoreMesh(core_axis_name="c", subcore_axis_name="s")

  @functools.partial(
      pl.kernel,
      out_type=jax.ShapeDtypeStruct((2, n_pad), jnp.float32),
      mesh=mesh,
      scratch_types=[
          pltpu.VMEM((K, 2, CH), jnp.int32),
          pltpu.VMEM((CH,), jnp.float32),
          pltpu.VMEM_SHARED((n_pad,), jnp.float32),
          pltpu.SemaphoreType.DMA,
      ],
      compiler_params=pltpu.CompilerParams(use_tc_tiling_on_sc=False),
  )
  def deg_pass(idx_hbm, zeros_hbm, out_hbm, blk, ones, acc, ssem):
    c = lax.axis_index("c")
    s = lax.axis_index("s")
    wid = s * 2 + c
    r0 = s * slice_per_tile
    for j in range(CH // 16):
      ones[pl.ds(j * 16, 16)] = jnp.ones((16,), jnp.float32)
    pltpu.sync_copy(zeros_hbm.at[pl.ds(r0, slice_per_tile)],
                    acc.at[pl.ds(r0, slice_per_tile)])
    plsc.subcore_barrier()

    base_row = wid * nch

    def body(bi, carry):
      row0 = base_row + bi * K
      pltpu.sync_copy(idx_hbm.at[pl.ds(row0, K)], blk)
      scatters = [
          pltpu.async_copy(ones, acc.at[blk.at[j, 1]], ssem, add=True)
          for j in range(K)
      ]
      for d in scatters:
        d.wait()
      return carry

    lax.fori_loop(0, nblk, body, 0)
    plsc.subcore_barrier()
    pltpu.sync_copy(acc.at[pl.ds(r0, slice_per_tile)],
                    out_hbm.at[c].at[pl.ds(r0, slice_per_tile)])

  return deg_pass


def _kdeg_body(p_ref, o_ref):
  deg = p_ref[0] + p_ref[1] + 1.0
  o_ref[...] = lax.rsqrt(jnp.maximum(deg, 1e-12))


def _k1_body(x_ref, w_ref, d_ref, o_ref):
  t = jnp.dot(x_ref[...], w_ref[...], preferred_element_type=jnp.float32)
  o_ref[...] = t * d_ref[...]


def _klayer_body(p_ref, tp_ref, d_ref, w_ref, b_ref, o_ref):
  d = d_ref[...]
  agg = d * (p_ref[0] + p_ref[1] + tp_ref[...])
  h = jnp.maximum(agg + b_ref[...], 0.0)
  o_ref[...] = jnp.dot(h, w_ref[...], preferred_element_type=jnp.float32) * d


def _klast_body(p_ref, tp_ref, d_ref, b_ref, o_ref):
  d = d_ref[...]
  o_ref[...] = d * (p_ref[0] + p_ref[1] + tp_ref[...]) + b_ref[...]


def _padW(w, rin, rout):
  return jnp.pad(w, ((0, rin - w.shape[0]), (0, rout - w.shape[1])))


def _padb(b, r):
  return jnp.pad(b, (0, r - b.shape[0]))


def kernel(x, edge_index, W1, b1, W2, b2, W3, b3, W4, b4, W5, b5,
           W6, b6, W7, b7, W8, b8, W9, b9, W10, b10):
  n = x.shape[0]
  e = edge_index.shape[1]
  grain = NTILES * CH * K
  e_pad = ((e + grain - 1) // grain) * grain
  n_pad = ((n + 2047) // 2048) * 2048
  nr = n_pad * F // 128

  src = edge_index[0]
  dst = edge_index[1]
  pad = e_pad - e
  if pad:
    src = jnp.concatenate([src, jnp.zeros((pad,), jnp.int32)])
    dst = jnp.concatenate([dst, jnp.full((pad,), n, jnp.int32)])
  idxg = jnp.stack([src.reshape(e_pad // CH, CH),
                    dst.reshape(e_pad // CH, CH)], axis=1)
  zeros8 = jnp.zeros((n_pad, F), jnp.float32)
  zeros1 = jnp.zeros((n_pad,), jnp.float32)

  deg_pass = _deg_pass_kernel(n_pad, e_pad)
  edge_pass = _edge_pass_kernel(n_pad, e_pad)

  degp = deg_pass(idxg, zeros1)
  dinv = pl.pallas_call(
      _kdeg_body,
      out_shape=jax.ShapeDtypeStruct((n_pad // 128, 128), jnp.float32),
  )(degp.reshape(2, n_pad // 128, 128))
  dexp = jnp.repeat(dinv.reshape(-1), F)
  d128 = dexp.reshape(nr, 128)

  xp = jnp.pad(x, ((0, n_pad - n), (0, 16 - x.shape[1])))
  Wbd1 = jnp.kron(jnp.eye(8, dtype=jnp.float32), _padW(W1, 16, F))
  tp = pl.pallas_call(
      _k1_body,
      out_shape=jax.ShapeDtypeStruct((n_pad // 8, 8 * F), jnp.float32),
  )(xp.reshape(n_pad // 8, 128), Wbd1, dexp.reshape(n_pad // 8, 8 * F))
  tp = tp.reshape(n_pad, F)

  Ws = [W2, W3, W4, W5, W6, W7, W8, W9, W10]
  bs = [b1, b2, b3, b4, b5, b6, b7, b8, b9]
  eye16 = jnp.eye(16, dtype=jnp.float32)
  for i in range(9):
    p = edge_pass(tp, idxg, zeros8)
    Wbd = jnp.kron(eye16, _padW(Ws[i], F, F))
    bt = jnp.tile(_padb(bs[i], F), 16).reshape(1, 128)
    tp = pl.pallas_call(
        _klayer_body,
        out_shape=jax.ShapeDtypeStruct((nr, 128), jnp.float32),
    )(p.reshape(2, nr, 128), tp.reshape(nr, 128), d128, Wbd, bt)
    tp = tp.reshape(n_pad, F)

  p = edge_pass(tp, idxg, zeros8)
  bt10 = jnp.tile(_padb(b10, F), 16).reshape(1, 128)
  out = pl.pallas_call(
      _klast_body,
      out_shape=jax.ShapeDtypeStruct((nr, 128), jnp.float32),
  )(p.reshape(2, nr, 128), tp.reshape(nr, 128), d128, bt10)
  return out.reshape(n_pad, F)[:n, :1]
